# Initial kernel scaffold; baseline (speedup 1.0000x reference)
import jax
import jax.numpy as jnp
from jax.experimental import pallas as pl


def kernel(nodes, neigh_idx, feat_table, W):
    raise NotImplementedError("write your pallas kernel here")


# speedup vs baseline: 3.5577x; 3.5577x over previous
import functools

import jax
import jax.numpy as jnp
from jax import lax
from jax.experimental import pallas as pl
from jax.experimental.pallas import tpu as pltpu
from jax.experimental.pallas import tpu_sc as plsc

_NC = 2
_NS = 16
_NW = _NC * _NS

_D = 128
_K = 10


def _build_sc_gather(b_pad: int, n_nodes: int, chunk: int):
    b_per_w = b_pad // _NW
    assert b_per_w % chunk == 0
    n_chunks = b_per_w // chunk

    mesh = plsc.VectorSubcoreMesh(core_axis_name="c", subcore_axis_name="s")

    @functools.partial(
        pl.kernel,
        mesh=mesh,
        out_type=(
            jax.ShapeDtypeStruct((b_pad, _D), jnp.float32),
            jax.ShapeDtypeStruct((b_pad, _D), jnp.float32),
        ),
        scratch_types=[
            pltpu.VMEM((chunk,), jnp.int32),
            pltpu.VMEM((chunk, _D), jnp.float32),
            pltpu.VMEM((chunk * _K,), jnp.int32),
            pltpu.VMEM((chunk * _K, _D), jnp.float32),
            pltpu.VMEM((chunk, _D), jnp.float32),
            pltpu.SemaphoreType.DMA,
        ],
    )
    def sc_gather(nodes_hbm, neigh_hbm, table_hbm, self_out, nsum_out,
                  sidx_v, srows_v, nidx_v, nrows_v, nsum_v, sem):
        wid = lax.axis_index("s") * _NC + lax.axis_index("c")
        base = wid * b_per_w

        @pl.loop(0, n_chunks)
        def _chunk_loop(g):
            off = base + g * chunk
            off10 = off * _K

            pltpu.sync_copy(nodes_hbm.at[pl.ds(off, chunk)], sidx_v)
            pltpu.async_copy(table_hbm.at[sidx_v], srows_v, sem).wait()
            pltpu.sync_copy(srows_v, self_out.at[pl.ds(off, chunk)])

            pltpu.sync_copy(neigh_hbm.at[pl.ds(off10, chunk * _K)], nidx_v)
            pltpu.async_copy(table_hbm.at[nidx_v], nrows_v, sem).wait()

            @pl.loop(0, chunk)
            def _node_loop(i):
                r0 = i * _K
                for c in range(_D // 16):
                    sl = pl.ds(c * 16, 16)
                    acc = nrows_v[r0, sl]
                    for j in range(1, _K):
                        acc = acc + nrows_v[r0 + j, sl]
                    nsum_v[i, sl] = acc

            pltpu.sync_copy(nsum_v, nsum_out.at[pl.ds(off, chunk)])

    return sc_gather


def _tc_matmul_body(s_ref, n_ref, w_ref, o_ref):
    s = s_ref[...]
    n = n_ref[...] * (1.0 / _K)
    acc = jnp.dot(s, w_ref[0:_D, :], preferred_element_type=jnp.float32)
    acc = acc + jnp.dot(n, w_ref[_D:2 * _D, :], preferred_element_type=jnp.float32)
    o_ref[...] = jnp.maximum(acc, 0.0)


def _tc_matmul(self_rows, nsum, w, bm: int):
    b_pad = self_rows.shape[0]
    grid = (b_pad // bm,)
    return pl.pallas_call(
        _tc_matmul_body,
        grid=grid,
        in_specs=[
            pl.BlockSpec((bm, _D), lambda i: (i, 0)),
            pl.BlockSpec((bm, _D), lambda i: (i, 0)),
            pl.BlockSpec((2 * _D, _D), lambda i: (0, 0)),
        ],
        out_specs=pl.BlockSpec((bm, _D), lambda i: (i, 0)),
        out_shape=jax.ShapeDtypeStruct((b_pad, _D), jnp.float32),
    )(self_rows, nsum, w)


def kernel(nodes, neigh_idx, feat_table, W):
    b = nodes.shape[0]
    n_nodes = feat_table.shape[0]

    chunk = 56
    unit = _NW * chunk
    b_pad = ((b + unit - 1) // unit) * unit
    pad = b_pad - b

    nodes_p = jnp.pad(nodes, (0, pad))
    neigh_flat = jnp.pad(neigh_idx.reshape(-1), (0, pad * _K))

    sc = _build_sc_gather(b_pad, n_nodes, chunk)
    self_rows, nsum = sc(nodes_p, neigh_flat, feat_table)

    out = _tc_matmul(self_rows, nsum, W, bm=1024)
    return out[:b]
